# baseline (speedup 1.0000x reference)
import functools

import jax
import jax.numpy as jnp
from jax import lax
from jax.experimental import pallas as pl
from jax.experimental.pallas import tpu as pltpu
from jax.experimental.pallas import tpu_sc as plsc

NUM_EMB = 1000000
DIM = 64
B_TOTAL = 4096 * 50
NUM_CORES = 2
NUM_SUBCORES = 16
NW = NUM_CORES * NUM_SUBCORES
B_PER_W = B_TOTAL // NW
CHUNK = 128
NCHUNK = B_PER_W // CHUNK

_MAGIC = 12582912.0
_INV_SCALE = 1024.0
_SCALE = 1.0 / 1024.0


def _quantize_buf(rows):
    def qrow(i, carry):
        for jj in range(DIM // 16):
            v = rows[i, pl.ds(jj * 16, 16)]
            r = (v * _INV_SCALE + _MAGIC) - _MAGIC
            q = jnp.minimum(jnp.maximum(r, -128.0), 127.0) * _SCALE
            rows[i, pl.ds(jj * 16, 16)] = q
        return carry
    lax.fori_loop(0, CHUNK, qrow, 0)


def _sc_body(weight_hbm, idx_hbm, out_hbm, idx_v, rows0, rows1, sem0, sem1):
    cid = lax.axis_index("c")
    sid = lax.axis_index("s")
    wid = sid * NUM_CORES + cid
    base = wid * B_PER_W

    pltpu.sync_copy(idx_hbm.at[wid], idx_v)

    bufs = (rows0, rows1)
    sems = (sem0, sem1)

    pltpu.async_copy(weight_hbm.at[idx_v.at[0]], rows0, sem0)

    def step(jh, carry):
        for p in range(2):
            j = jh * 2 + p
            buf, sem = bufs[p], sems[p]
            nbuf, nsem = bufs[1 - p], sems[1 - p]
            pltpu.make_async_copy(weight_hbm.at[idx_v.at[j]], buf, sem).wait()
            @pl.when(j + 1 < NCHUNK)
            def _():
                pltpu.async_copy(weight_hbm.at[idx_v.at[j + 1]], nbuf, nsem)
            _quantize_buf(buf)
            pltpu.sync_copy(buf, out_hbm.at[pl.ds(base + j * CHUNK, CHUNK)])
        return carry

    lax.fori_loop(0, NCHUNK // 2, step, 0)


@jax.jit
def _sc_lookup(idx_grouped, weight):
    mesh = plsc.VectorSubcoreMesh(core_axis_name="c", subcore_axis_name="s")
    k = functools.partial(
        pl.kernel,
        mesh=mesh,
        out_type=jax.ShapeDtypeStruct((B_TOTAL, DIM), jnp.float32),
        scratch_types=[
            pltpu.VMEM((NCHUNK, CHUNK), jnp.int32),
            pltpu.VMEM((CHUNK, DIM), jnp.float32),
            pltpu.VMEM((CHUNK, DIM), jnp.float32),
            pltpu.SemaphoreType.DMA,
            pltpu.SemaphoreType.DMA,
        ],
        compiler_params=pltpu.CompilerParams(use_tc_tiling_on_sc=False),
    )(_sc_body)
    return k(weight, idx_grouped)


def kernel(x, weight):
    idx = x.reshape(NW, NCHUNK, CHUNK).astype(jnp.int32)
    out = _sc_lookup(idx, weight)
    return out.reshape(x.shape[0], x.shape[1], DIM)


# speedup vs baseline: 1.3986x; 1.3986x over previous
import functools

import jax
import jax.numpy as jnp
from jax import lax
from jax.experimental import pallas as pl
from jax.experimental.pallas import tpu as pltpu
from jax.experimental.pallas import tpu_sc as plsc

NUM_EMB = 1000000
DIM = 64
NROW = 4096
NSLOT = 50
NUM_CORES = 2
NUM_SUBCORES = 16
NW = NUM_CORES * NUM_SUBCORES
RBLK = NROW // NW
NBUF = 4
NSLOT_MAIN = (NSLOT // NBUF) * NBUF

_INV_SCALE = 1024.0
_SCALE = 1.0 / 1024.0

BN = 8192


def _quant_tc_body(wt_ref, out_ref):
    w = wt_ref[...]
    q = jnp.clip(jnp.round(w * _INV_SCALE), -128.0, 127.0) * _SCALE
    out_ref[...] = jnp.concatenate(
        [q.T, jnp.zeros((BN, DIM), jnp.float32)], axis=1)


@jax.jit
def _quantize_table(wt):
    grid = (NUM_EMB + BN - 1) // BN
    return pl.pallas_call(
        _quant_tc_body,
        grid=(grid,),
        in_specs=[pl.BlockSpec((DIM, BN), lambda i: (0, i))],
        out_specs=pl.BlockSpec((BN, 2 * DIM), lambda i: (i, 0)),
        out_shape=jax.ShapeDtypeStruct((NUM_EMB, 2 * DIM), jnp.float32),
    )(wt)


def _transpose_chunk(gbuf, obuf):
    iota = lax.iota(jnp.int32, 16)

    def row(i, carry):
        col = jnp.full((16,), i, jnp.int32)
        for b in range(4):
            v = gbuf[i, pl.ds(16 * b, 16)]
            plsc.store_scatter(obuf, [iota + 16 * b, col], v)
        return carry

    lax.fori_loop(0, RBLK, row, 0)


def _sc_body(qt_hbm, xt_hbm, out_hbm, idx_v, gbufs, obufs, gsems, osems):
    cid = lax.axis_index("c")
    sid = lax.axis_index("s")
    wid = sid * NUM_CORES + cid
    rbase = wid * RBLK

    pltpu.sync_copy(xt_hbm.at[:, pl.ds(rbase, RBLK)], idx_v)

    def gather(s, b):
        pltpu.async_copy(qt_hbm.at[idx_v.at[s]], gbufs[b], gsems[b])

    def wait_gather(s, b):
        pltpu.make_async_copy(
            qt_hbm.at[idx_v.at[s]], gbufs[b], gsems[b]).wait()

    def put(s, b):
        pltpu.async_copy(
            obufs[b], out_hbm.at[s, :, pl.ds(rbase, RBLK)], osems[b])

    def wait_put(s, b):
        pltpu.make_async_copy(
            obufs[b], out_hbm.at[s, :, pl.ds(rbase, RBLK)], osems[b]).wait()

    for b in range(NBUF):
        gather(b, b)

    def step(jh, carry):
        for p in range(NBUF):
            s = jh * NBUF + p
            wait_gather(s, p)
            @pl.when(jh > 0)
            def _():
                wait_put(s, p)
            _transpose_chunk(gbufs[p], obufs[p])
            @pl.when(s + NBUF < NSLOT)
            def _():
                gather(s + NBUF, p)
            put(s, p)
        return carry

    lax.fori_loop(0, NSLOT_MAIN // NBUF, step, 0)

    for s in range(NSLOT_MAIN, NSLOT):
        p = s - NSLOT_MAIN
        wait_gather(s, p)
        wait_put(s, p)
        _transpose_chunk(gbufs[p], obufs[p])
        put(s, p)

    for p in range(NBUF):
        wait_put(0, p)


@jax.jit
def _sc_lookup(xt, qt):
    mesh = plsc.VectorSubcoreMesh(core_axis_name="c", subcore_axis_name="s")
    k = functools.partial(
        pl.kernel,
        mesh=mesh,
        out_type=jax.ShapeDtypeStruct((NSLOT, DIM, NROW), jnp.float32),
        scratch_types=[
            pltpu.VMEM((NSLOT, RBLK), jnp.int32),
            [pltpu.VMEM((RBLK, 2 * DIM), jnp.float32) for _ in range(NBUF)],
            [pltpu.VMEM((DIM, RBLK), jnp.float32) for _ in range(NBUF)],
            [pltpu.SemaphoreType.DMA for _ in range(NBUF)],
            [pltpu.SemaphoreType.DMA for _ in range(NBUF)],
        ],
        compiler_params=pltpu.CompilerParams(
            use_tc_tiling_on_sc=False, needs_layout_passes=False),
    )(_sc_body)
    return k(qt, xt)


def kernel(x, weight):
    qt = _quantize_table(weight.T)
    xt = x.T.astype(jnp.int32)
    out_phys = _sc_lookup(xt, qt)
    return out_phys.transpose(2, 0, 1)


# speedup vs baseline: 1.4039x; 1.0038x over previous
import functools

import jax
import jax.numpy as jnp
from jax import lax
from jax.experimental import pallas as pl
from jax.experimental.pallas import tpu as pltpu
from jax.experimental.pallas import tpu_sc as plsc

NUM_EMB = 1000000
DIM = 64
NROW = 4096
NSLOT = 50
NUM_CORES = 2
NUM_SUBCORES = 16
NW = NUM_CORES * NUM_SUBCORES
RBLK = NROW // NW
NBUF = 5
NSLOT_MAIN = (NSLOT // NBUF) * NBUF

_INV_SCALE = 1024.0
_SCALE = 1.0 / 1024.0

BN = 8192


def _quant_tc_body(wt_ref, out_ref):
    w = wt_ref[...]
    q = jnp.clip(jnp.round(w * _INV_SCALE), -128.0, 127.0) * _SCALE
    out_ref[...] = jnp.concatenate(
        [q.T, jnp.zeros((BN, DIM), jnp.float32)], axis=1)


@jax.jit
def _quantize_table(wt):
    grid = (NUM_EMB + BN - 1) // BN
    return pl.pallas_call(
        _quant_tc_body,
        grid=(grid,),
        in_specs=[pl.BlockSpec((DIM, BN), lambda i: (0, i))],
        out_specs=pl.BlockSpec((BN, 2 * DIM), lambda i: (i, 0)),
        out_shape=jax.ShapeDtypeStruct((NUM_EMB, 2 * DIM), jnp.float32),
    )(wt)


def _transpose_chunk(gbuf, obuf):
    iota = lax.iota(jnp.int32, 16)

    def row(i, carry):
        col = jnp.full((16,), i, jnp.int32)
        for b in range(4):
            v = gbuf[i, pl.ds(16 * b, 16)]
            plsc.store_scatter(obuf, [iota + 16 * b, col], v)
        return carry

    lax.fori_loop(0, RBLK, row, 0)


def _sc_body(qt_hbm, xt_hbm, out_hbm, idx_v, gbufs, obufs, gsems, osems):
    cid = lax.axis_index("c")
    sid = lax.axis_index("s")
    wid = sid * NUM_CORES + cid
    rbase = wid * RBLK

    pltpu.sync_copy(xt_hbm.at[:, pl.ds(rbase, RBLK)], idx_v)

    def gather(s, b):
        pltpu.async_copy(qt_hbm.at[idx_v.at[s]], gbufs[b], gsems[b])

    def wait_gather(s, b):
        pltpu.make_async_copy(
            qt_hbm.at[idx_v.at[s]], gbufs[b], gsems[b]).wait()

    def put(s, b):
        pltpu.async_copy(
            obufs[b], out_hbm.at[s, :, pl.ds(rbase, RBLK)], osems[b])

    def wait_put(s, b):
        pltpu.make_async_copy(
            obufs[b], out_hbm.at[s, :, pl.ds(rbase, RBLK)], osems[b]).wait()

    for b in range(NBUF):
        gather(b, b)

    def step(jh, carry):
        for p in range(NBUF):
            s = jh * NBUF + p
            wait_gather(s, p)
            @pl.when(jh > 0)
            def _():
                wait_put(s, p)
            _transpose_chunk(gbufs[p], obufs[p])
            @pl.when(s + NBUF < NSLOT)
            def _():
                gather(s + NBUF, p)
            put(s, p)
        return carry

    lax.fori_loop(0, NSLOT_MAIN // NBUF, step, 0)

    for p in range(NBUF):
        wait_put(0, p)


@jax.jit
def _sc_lookup(xt, qt):
    mesh = plsc.VectorSubcoreMesh(core_axis_name="c", subcore_axis_name="s")
    k = functools.partial(
        pl.kernel,
        mesh=mesh,
        out_type=jax.ShapeDtypeStruct((NSLOT, DIM, NROW), jnp.float32),
        scratch_types=[
            pltpu.VMEM((NSLOT, RBLK), jnp.int32),
            [pltpu.VMEM((RBLK, DIM), jnp.float32) for _ in range(NBUF)],
            [pltpu.VMEM((DIM, RBLK), jnp.float32) for _ in range(NBUF)],
            [pltpu.SemaphoreType.DMA for _ in range(NBUF)],
            [pltpu.SemaphoreType.DMA for _ in range(NBUF)],
        ],
        compiler_params=pltpu.CompilerParams(
            use_tc_tiling_on_sc=False, needs_layout_passes=False),
    )(_sc_body)
    return k(qt, xt)


def kernel(x, weight):
    qt = _quantize_table(weight.T)
    qt2 = qt.reshape(2 * NUM_EMB, DIM)
    xt2 = x.T.astype(jnp.int32) * 2
    out_phys = _sc_lookup(xt2, qt2)
    return out_phys.transpose(2, 0, 1)


# speedup vs baseline: 2.2078x; 1.5726x over previous
import functools

import jax
import jax.numpy as jnp
from jax import lax
from jax.experimental import pallas as pl
from jax.experimental.pallas import tpu as pltpu
from jax.experimental.pallas import tpu_sc as plsc

NUM_EMB = 1000000
DIM = 64
NROW = 4096
NSLOT = 50
NUM_CORES = 2
NUM_SUBCORES = 16
NW = NUM_CORES * NUM_SUBCORES
RBLK = NROW // NW
NBUF = 5
NSLOT_MAIN = (NSLOT // NBUF) * NBUF

_INV_SCALE = 1024.0
_SCALE = 1.0 / 1024.0

BN = 8192


def _quant_tc_body(wt_ref, out_ref):
    w = wt_ref[...]
    q = jnp.clip(jnp.round(w * _INV_SCALE), -128.0, 127.0) * _SCALE
    out_ref[...] = jnp.concatenate(
        [q.T, jnp.zeros((BN, DIM), jnp.float32)], axis=1)


@jax.jit
def _quantize_table(wt):
    grid = (NUM_EMB + BN - 1) // BN
    return pl.pallas_call(
        _quant_tc_body,
        grid=(grid,),
        in_specs=[pl.BlockSpec((DIM, BN), lambda i: (0, i))],
        out_specs=pl.BlockSpec((BN, 2 * DIM), lambda i: (i, 0)),
        out_shape=jax.ShapeDtypeStruct((NUM_EMB, 2 * DIM), jnp.float32),
    )(wt)


def _transpose_chunk(gbuf, obuf):
    iota = lax.iota(jnp.int32, 16)
    rows16 = [iota + 16 * b for b in range(4)]

    @functools.partial(plsc.parallel_loop, 0, RBLK, unroll=8)
    def _(i):
        col = jnp.full((16,), i, jnp.int32)
        for b in range(4):
            v = gbuf[i, pl.ds(16 * b, 16)]
            plsc.store_scatter(obuf, [rows16[b], col], v)


def _sc_body(qt_hbm, xt_hbm, out_hbm, idx_v, gbufs, obufs, gsems, osems):
    cid = lax.axis_index("c")
    sid = lax.axis_index("s")
    wid = sid * NUM_CORES + cid
    rbase = wid * RBLK

    pltpu.sync_copy(xt_hbm.at[:, pl.ds(rbase, RBLK)], idx_v)

    def gather(s, b):
        pltpu.async_copy(qt_hbm.at[idx_v.at[s]], gbufs[b], gsems[b])

    def wait_gather(s, b):
        pltpu.make_async_copy(
            qt_hbm.at[idx_v.at[s]], gbufs[b], gsems[b]).wait()

    def put(s, b):
        pltpu.async_copy(
            obufs[b], out_hbm.at[s, :, pl.ds(rbase, RBLK)], osems[b])

    def wait_put(s, b):
        pltpu.make_async_copy(
            obufs[b], out_hbm.at[s, :, pl.ds(rbase, RBLK)], osems[b]).wait()

    for b in range(NBUF):
        gather(b, b)

    def step(jh, carry):
        for p in range(NBUF):
            s = jh * NBUF + p
            wait_gather(s, p)
            @pl.when(jh > 0)
            def _():
                wait_put(s, p)
            _transpose_chunk(gbufs[p], obufs[p])
            @pl.when(s + NBUF < NSLOT)
            def _():
                gather(s + NBUF, p)
            put(s, p)
        return carry

    lax.fori_loop(0, NSLOT_MAIN // NBUF, step, 0)

    for p in range(NBUF):
        wait_put(0, p)


@jax.jit
def _sc_lookup(xt, qt):
    mesh = plsc.VectorSubcoreMesh(core_axis_name="c", subcore_axis_name="s")
    k = functools.partial(
        pl.kernel,
        mesh=mesh,
        out_type=jax.ShapeDtypeStruct((NSLOT, DIM, NROW), jnp.float32),
        scratch_types=[
            pltpu.VMEM((NSLOT, RBLK), jnp.int32),
            [pltpu.VMEM((RBLK, DIM), jnp.float32) for _ in range(NBUF)],
            [pltpu.VMEM((DIM, RBLK), jnp.float32) for _ in range(NBUF)],
            [pltpu.SemaphoreType.DMA for _ in range(NBUF)],
            [pltpu.SemaphoreType.DMA for _ in range(NBUF)],
        ],
        compiler_params=pltpu.CompilerParams(
            use_tc_tiling_on_sc=False, needs_layout_passes=False),
    )(_sc_body)
    return k(qt, xt)


def kernel(x, weight):
    qt = _quantize_table(weight.T)
    qt2 = qt.reshape(2 * NUM_EMB, DIM)
    xt2 = x.T.astype(jnp.int32) * 2
    out_phys = _sc_lookup(xt2, qt2)
    return out_phys.transpose(2, 0, 1)


# speedup vs baseline: 2.5719x; 1.1649x over previous
import functools

import jax
import jax.numpy as jnp
from jax import lax
from jax.experimental import pallas as pl
from jax.experimental.pallas import tpu as pltpu
from jax.experimental.pallas import tpu_sc as plsc

NUM_EMB = 1000000
DIM = 64
NROW = 4096
NSLOT = 50
NUM_CORES = 2
NUM_SUBCORES = 16
NW = NUM_CORES * NUM_SUBCORES
RBLK = NROW // NW
NBUF = 5
NSLOT_MAIN = (NSLOT // NBUF) * NBUF

_INV_SCALE = 1024.0
_SCALE = 1.0 / 1024.0

BN = 8192


def _quant_tc_body(wt_ref, out_ref):
    w = wt_ref[...]
    q = jnp.clip(jnp.round(w * _INV_SCALE), -128.0, 127.0) * _SCALE
    out_ref[...] = jnp.concatenate(
        [q.T, jnp.zeros((BN, DIM), jnp.float32)], axis=1)


@jax.jit
def _quantize_table(wt):
    grid = (NUM_EMB + BN - 1) // BN
    return pl.pallas_call(
        _quant_tc_body,
        grid=(grid,),
        in_specs=[pl.BlockSpec((DIM, BN), lambda i: (0, i))],
        out_specs=pl.BlockSpec((BN, 2 * DIM), lambda i: (i, 0)),
        out_shape=jax.ShapeDtypeStruct((NUM_EMB, 2 * DIM), jnp.float32),
    )(wt)


def _transpose_chunk(gbuf, obuf):
    iota = lax.iota(jnp.int32, 16)
    d16 = [iota + 16 * b for b in range(4)]
    dtile = [lax.shift_right_logical(d, 3) for d in d16]
    dbase = [lax.shift_left(jnp.bitwise_and(d, 7), 7) for d in d16]

    @functools.partial(plsc.parallel_loop, 0, RBLK, unroll=8)
    def _(i):
        for b in range(4):
            v = gbuf[i, pl.ds(16 * b, 16)]
            plsc.store_scatter(obuf, [dtile[b], dbase[b] + i], v)


def _sc_body(qt_hbm, xt_hbm, out_hbm, idx_v, gbufs, obufs, gsems, osems):
    cid = lax.axis_index("c")
    sid = lax.axis_index("s")
    wid = sid * NUM_CORES + cid
    rbase = wid * RBLK

    pltpu.sync_copy(xt_hbm.at[:, pl.ds(rbase, RBLK)], idx_v)

    def gather(s, b):
        pltpu.async_copy(qt_hbm.at[idx_v.at[s]], gbufs[b], gsems[b])

    def wait_gather(s, b):
        pltpu.make_async_copy(
            qt_hbm.at[idx_v.at[s]], gbufs[b], gsems[b]).wait()

    def put(s, b):
        pltpu.async_copy(obufs[b], out_hbm.at[s, :, wid, :], osems[b])

    def wait_put(s, b):
        pltpu.make_async_copy(
            obufs[b], out_hbm.at[s, :, wid, :], osems[b]).wait()

    for b in range(NBUF):
        gather(b, b)

    def step(jh, carry):
        for p in range(NBUF):
            s = jh * NBUF + p
            wait_gather(s, p)
            @pl.when(jh > 0)
            def _():
                wait_put(s, p)
            _transpose_chunk(gbufs[p], obufs[p])
            @pl.when(s + NBUF < NSLOT)
            def _():
                gather(s + NBUF, p)
            put(s, p)
        return carry

    lax.fori_loop(0, NSLOT_MAIN // NBUF, step, 0)

    for p in range(NBUF):
        wait_put(0, p)


@jax.jit
def _sc_lookup(xt, qt):
    mesh = plsc.VectorSubcoreMesh(core_axis_name="c", subcore_axis_name="s")
    k = functools.partial(
        pl.kernel,
        mesh=mesh,
        out_type=jax.ShapeDtypeStruct((NSLOT, DIM // 8, NW, 8 * RBLK),
                                      jnp.float32),
        scratch_types=[
            pltpu.VMEM((NSLOT, RBLK), jnp.int32),
            [pltpu.VMEM((RBLK, DIM), jnp.float32) for _ in range(NBUF)],
            [pltpu.VMEM((DIM // 8, 8 * RBLK), jnp.float32)
             for _ in range(NBUF)],
            [pltpu.SemaphoreType.DMA for _ in range(NBUF)],
            [pltpu.SemaphoreType.DMA for _ in range(NBUF)],
        ],
        compiler_params=pltpu.CompilerParams(
            use_tc_tiling_on_sc=False, needs_layout_passes=False),
    )(_sc_body)
    return k(qt, xt)


def kernel(x, weight):
    qt = _quantize_table(weight.T)
    qt2 = qt.reshape(2 * NUM_EMB, DIM)
    xt2 = x.T.astype(jnp.int32) * 2
    out4d = _sc_lookup(xt2, qt2)
    t = out4d.reshape(NSLOT, 8, NW, 8, RBLK)
    return t.transpose(2, 4, 0, 1, 3).reshape(NROW, NSLOT, DIM)
